# baseline (speedup 1.0000x reference)
import functools

import jax
import jax.numpy as jnp
from jax import lax
from jax.experimental import pallas as pl
from jax.experimental.pallas import tpu as pltpu


def _mha_kernel(x0_ref, xn_ref, w_ref, bqkv_ref, wo_ref, ob_ref,
                out_ref, attn_ref, qs_ref, ks_ref, vs_ref, *,
                g, dk, nj, n_steps):
    n = pl.program_id(0)
    j = lax.rem(n, nj)
    jn = lax.rem(n + 1, nj)
    par = lax.rem(n, 2)
    gd = g * dk
    D = x0_ref.shape[2]

    def proj_group(x_bf, base, dst_ref, slot):
        acc = jnp.dot(x_bf, w_ref[:, pl.ds(base, gd)],
                      preferred_element_type=jnp.float32)
        dst_ref[slot] = (acc + bqkv_ref[:, pl.ds(base, gd)]).astype(jnp.bfloat16)

    @pl.when(n == 0)
    def _prime():
        xb = x0_ref[0].astype(jnp.bfloat16)
        proj_group(xb, 0, qs_ref, 0)
        proj_group(xb, D, ks_ref, 0)
        proj_group(xb, 2 * D, vs_ref, 0)

    @pl.when(n + 1 < n_steps)
    def _next_proj():
        xb = xn_ref[0].astype(jnp.bfloat16)
        proj_group(xb, jn * gd, qs_ref, 1 - par)
        proj_group(xb, D + jn * gd, ks_ref, 1 - par)
        proj_group(xb, 2 * D + jn * gd, vs_ref, 1 - par)

    q = qs_ref[par]
    k = ks_ref[par]
    v = vs_ref[par]

    ctx_parts = []
    for h in range(g):
        sl = slice(h * dk, (h + 1) * dk)
        s = lax.dot_general(q[:, sl], k[:, sl], (((1,), (1,)), ((), ())),
                            preferred_element_type=jnp.float32)
        row_max = jnp.max(s, axis=-1, keepdims=True)
        denom = jnp.sum(jnp.exp(s - row_max), axis=-1, keepdims=True)
        p = jnp.exp(s - (row_max + jnp.log(denom)))
        attn_ref[0, h] = p
        ctx_parts.append(jnp.dot(p.astype(jnp.bfloat16), v[:, sl],
                                 preferred_element_type=jnp.float32))

    ctx = jnp.concatenate(ctx_parts, axis=1).astype(jnp.bfloat16)
    wo = wo_ref[pl.ds(j * gd, gd), :]
    partial = jnp.dot(ctx, wo, preferred_element_type=jnp.float32)

    @pl.when(j == 0)
    def _init():
        out_ref[0] = partial + ob_ref[...]

    @pl.when(j != 0)
    def _acc():
        out_ref[0] += partial


def kernel(x, qkv_wt, qkv_b, o_wt, o_b):
    bs, L, D = x.shape
    dk = 64
    nh = D // dk
    g = 8
    nj = nh // g
    gd = g * dk
    n_steps = bs * nj

    b2 = qkv_b.reshape(1, 3 * D).astype(jnp.float32)
    ob2 = o_b.reshape(1, D).astype(jnp.float32)

    def bmap(n):
        return lax.div(n, nj)

    def bnext(n):
        return lax.div(jnp.minimum(n + 1, n_steps - 1), nj)

    out, attn = pl.pallas_call(
        functools.partial(_mha_kernel, g=g, dk=dk, nj=nj, n_steps=n_steps),
        out_shape=(
            jax.ShapeDtypeStruct((bs, L, D), jnp.float32),
            jax.ShapeDtypeStruct((bs, nh, L, L), jnp.float32),
        ),
        grid=(n_steps,),
        in_specs=[
            pl.BlockSpec((1, L, D), lambda n: (0, 0, 0)),
            pl.BlockSpec((1, L, D), lambda n: (bnext(n), 0, 0)),
            pl.BlockSpec((D, 3 * D), lambda n: (0, 0)),
            pl.BlockSpec((1, 3 * D), lambda n: (0, 0)),
            pl.BlockSpec((D, D), lambda n: (0, 0)),
            pl.BlockSpec((1, D), lambda n: (0, 0)),
        ],
        out_specs=(
            pl.BlockSpec((1, L, D), lambda n: (bmap(n), 0, 0)),
            pl.BlockSpec((1, g, L, L), lambda n: (bmap(n), lax.rem(n, nj), 0, 0)),
        ),
        scratch_shapes=[
            pltpu.VMEM((2, L, gd), jnp.bfloat16),
            pltpu.VMEM((2, L, gd), jnp.bfloat16),
            pltpu.VMEM((2, L, gd), jnp.bfloat16),
        ],
        compiler_params=pltpu.CompilerParams(
            dimension_semantics=("arbitrary",),
            vmem_limit_bytes=56 * 1024 * 1024,
        ),
    )(x, x, qkv_wt, b2, o_wt, ob2)
    return out, attn


# speedup vs baseline: 1.2831x; 1.2831x over previous
import functools

import jax
import jax.numpy as jnp
from jax import lax
from jax.experimental import pallas as pl
from jax.experimental.pallas import tpu as pltpu


def _mha_kernel(x_ref, w_ref, bqkv_ref, wo_ref, ob_ref,
                out_ref, attn_ref, *, g, dk):
    j = pl.program_id(1)
    gd = g * dk
    D = x_ref.shape[2]

    x = x_ref[0].astype(jnp.bfloat16)

    def proj_group(base):
        acc = jnp.dot(x, w_ref[:, pl.ds(base, gd)],
                      preferred_element_type=jnp.float32)
        return (acc + bqkv_ref[:, pl.ds(base, gd)]).astype(jnp.bfloat16)

    q = proj_group(j * gd)
    k = proj_group(D + j * gd)
    v = proj_group(2 * D + j * gd)

    ctx_parts = []
    for h in range(g):
        sl = slice(h * dk, (h + 1) * dk)
        s = lax.dot_general(q[:, sl], k[:, sl], (((1,), (1,)), ((), ())),
                            preferred_element_type=jnp.float32)
        row_max = jnp.max(s, axis=-1, keepdims=True)
        denom = jnp.sum(jnp.exp(s - row_max), axis=-1, keepdims=True)
        p = jnp.exp(s - (row_max + jnp.log(denom)))
        attn_ref[0, h] = p
        ctx_parts.append(jnp.dot(p.astype(jnp.bfloat16), v[:, sl],
                                 preferred_element_type=jnp.float32))

    ctx = jnp.concatenate(ctx_parts, axis=1).astype(jnp.bfloat16)
    wo = wo_ref[pl.ds(j * gd, gd), :]
    partial = jnp.dot(ctx, wo, preferred_element_type=jnp.float32)

    @pl.when(j == 0)
    def _init():
        out_ref[0] = partial + ob_ref[...]

    @pl.when(j != 0)
    def _acc():
        out_ref[0] += partial


def kernel(x, qkv_wt, qkv_b, o_wt, o_b):
    bs, L, D = x.shape
    dk = 64
    nh = D // dk
    g = 8
    nj = nh // g
    gd = g * dk

    b2 = qkv_b.reshape(1, 3 * D).astype(jnp.float32)
    ob2 = o_b.reshape(1, D).astype(jnp.float32)

    out, attn = pl.pallas_call(
        functools.partial(_mha_kernel, g=g, dk=dk),
        out_shape=(
            jax.ShapeDtypeStruct((bs, L, D), jnp.float32),
            jax.ShapeDtypeStruct((bs, nh, L, L), jnp.float32),
        ),
        grid=(bs, nj),
        in_specs=[
            pl.BlockSpec((1, L, D), lambda b, j: (b, 0, 0)),
            pl.BlockSpec((D, 3 * D), lambda b, j: (0, 0)),
            pl.BlockSpec((1, 3 * D), lambda b, j: (0, 0)),
            pl.BlockSpec((D, D), lambda b, j: (0, 0)),
            pl.BlockSpec((1, D), lambda b, j: (0, 0)),
        ],
        out_specs=(
            pl.BlockSpec((1, L, D), lambda b, j: (b, 0, 0)),
            pl.BlockSpec((1, g, L, L), lambda b, j: (b, j, 0, 0)),
        ),
        compiler_params=pltpu.CompilerParams(
            dimension_semantics=("parallel", "arbitrary"),
            vmem_limit_bytes=56 * 1024 * 1024,
        ),
    )(x, qkv_wt, b2, o_wt, ob2)
    return out, attn
